# baseline (speedup 1.0000x reference)
import jax
import jax.numpy as jnp
from jax.experimental import pallas as pl
from jax.experimental.pallas import tpu as pltpu


def _mlp_kernel(x_ref, w1_ref, b1_ref, w2_ref, b2_ref, o_ref):
    f = pl.program_id(1)
    h = jnp.dot(x_ref[0], w1_ref[0], preferred_element_type=jnp.float32)
    h = jnp.maximum(h + b1_ref[0], 0.0)
    part = jnp.dot(h, w2_ref[0], preferred_element_type=jnp.float32)

    @pl.when(f == 0)
    def _init():
        o_ref[0] = part + b2_ref[0]

    @pl.when(f != 0)
    def _acc():
        o_ref[0] += part


def kernel(x, W1, b1, W2, b2):
    B, S, D = x.shape
    E, _, F = W1.shape
    T = B * S
    per = T // E
    BF = F // 2
    xr = x.reshape(E, per, D)
    out = pl.pallas_call(
        _mlp_kernel,
        grid=(E, F // BF),
        in_specs=[
            pl.BlockSpec((1, per, D), lambda e, f: (e, 0, 0)),
            pl.BlockSpec((1, D, BF), lambda e, f: (e, 0, f)),
            pl.BlockSpec((1, 1, BF), lambda e, f: (e, 0, f)),
            pl.BlockSpec((1, BF, D), lambda e, f: (e, f, 0)),
            pl.BlockSpec((1, 1, D), lambda e, f: (e, 0, 0)),
        ],
        out_specs=pl.BlockSpec((1, per, D), lambda e, f: (e, 0, 0)),
        out_shape=jax.ShapeDtypeStruct((E, per, D), x.dtype),
        compiler_params=pltpu.CompilerParams(
            dimension_semantics=("arbitrary", "arbitrary"),
        ),
    )(xr, W1, b1.reshape(E, 1, F), W2, b2.reshape(E, 1, D))
    return out.reshape(B, S, D)
